# Initial kernel scaffold; baseline (speedup 1.0000x reference)
import jax
import jax.numpy as jnp
from jax.experimental import pallas as pl


def kernel(gt_ab, q_ab):
    raise NotImplementedError("write your pallas kernel here")


# speedup vs baseline: 25.2729x; 25.2729x over previous
import jax
import jax.numpy as jnp
from jax.experimental import pallas as pl

_NBINS = 313
_K = 5
_INV_2SIG2 = 1.0 / 50.0
_ROWS = 8


def _bf16_round(x):
    return x.astype(jnp.bfloat16).astype(jnp.float32)


def _encode_kernel(x_ref, q_ref, o_ref):
    a = x_ref[0, 0]
    b = x_ref[0, 1]
    q = q_ref[...]
    qa = q[:, 0:1].reshape(_NBINS, 1, 1)
    qb = q[:, 1:2].reshape(_NBINS, 1, 1)

    q_sq = qa * qa + qb * qb
    p_sq = (a * a + b * b)[None]
    qp = _bf16_round(qa) * _bf16_round(a)[None] + _bf16_round(qb) * _bf16_round(b)[None]
    d2 = jnp.maximum((q_sq + p_sq) - 2.0 * qp, 0.0)

    iota = jax.lax.broadcasted_iota(jnp.int32, d2.shape, 0)
    masked = d2
    acc = jnp.zeros_like(d2)
    z = jnp.zeros_like(a)
    for _ in range(_K):
        mi = jnp.min(masked, axis=0)
        eq = masked == mi[None]
        idx = jnp.min(jnp.where(eq, iota, _NBINS), axis=0)
        onehot = iota == idx[None]
        w = jnp.exp(mi * (-_INV_2SIG2))
        acc = acc + jnp.where(onehot, w[None], 0.0)
        z = z + w
        masked = jnp.where(onehot, jnp.inf, masked)
    o_ref[0] = acc * (1.0 / z)[None]


def kernel(gt_ab, q_ab):
    bs, _, H, W = gt_ab.shape
    grid = (bs, H // _ROWS)
    return pl.pallas_call(
        _encode_kernel,
        grid=grid,
        in_specs=[
            pl.BlockSpec((1, 2, _ROWS, W), lambda i, j: (i, 0, j, 0)),
            pl.BlockSpec((_NBINS, 2), lambda i, j: (0, 0)),
        ],
        out_specs=pl.BlockSpec((1, _NBINS, _ROWS, W), lambda i, j: (i, 0, j, 0)),
        out_shape=jax.ShapeDtypeStruct((bs, _NBINS, H, W), gt_ab.dtype),
    )(gt_ab, q_ab)


# speedup vs baseline: 31.5904x; 1.2500x over previous
import jax
import jax.numpy as jnp
from jax.experimental import pallas as pl

_NBINS = 313
_K = 5
_INV_2SIG2 = 1.0 / 50.0
_ROWS = 8


def _bf16_round(x):
    return x.astype(jnp.bfloat16).astype(jnp.float32)


def _encode_kernel(x_ref, q_ref, o_ref):
    a = x_ref[0, 0]
    b = x_ref[0, 1]
    q = q_ref[...]
    qa = q[:, 0:1].reshape(_NBINS, 1, 1)
    qb = q[:, 1:2].reshape(_NBINS, 1, 1)

    q_sq = qa * qa + qb * qb
    p_sq = (a * a + b * b)[None]
    qp = _bf16_round(qa) * _bf16_round(a)[None] + _bf16_round(qb) * _bf16_round(b)[None]
    d2 = jnp.maximum((q_sq + p_sq) - 2.0 * qp, 0.0)

    iota = jax.lax.broadcasted_iota(jnp.int32, d2.shape, 0)
    keys = (jax.lax.bitcast_convert_type(d2, jnp.int32) & ~511) | iota
    acc = jnp.zeros_like(d2)
    z = jnp.zeros_like(a)
    for i in range(_K):
        mi = jnp.min(keys, axis=0)
        onehot = keys == mi[None]
        d2min = jax.lax.bitcast_convert_type(mi & ~511, jnp.float32)
        w = jnp.exp(d2min * (-_INV_2SIG2))
        acc = acc + jnp.where(onehot, w[None], 0.0)
        z = z + w
        if i + 1 < _K:
            keys = jnp.where(onehot, jnp.int32(0x7FFFFFFF), keys)
    o_ref[0] = acc * (1.0 / z)[None]


def kernel(gt_ab, q_ab):
    bs, _, H, W = gt_ab.shape
    grid = (bs, H // _ROWS)
    return pl.pallas_call(
        _encode_kernel,
        grid=grid,
        in_specs=[
            pl.BlockSpec((1, 2, _ROWS, W), lambda i, j: (i, 0, j, 0)),
            pl.BlockSpec((_NBINS, 2), lambda i, j: (0, 0)),
        ],
        out_specs=pl.BlockSpec((1, _NBINS, _ROWS, W), lambda i, j: (i, 0, j, 0)),
        out_shape=jax.ShapeDtypeStruct((bs, _NBINS, H, W), gt_ab.dtype),
    )(gt_ab, q_ab)
